# baseline (speedup 1.0000x reference)
import functools

import jax
import jax.numpy as jnp
from jax import lax
from jax.experimental import pallas as pl

_PREC = lax.Precision.HIGHEST
_F32 = jnp.float32


def _mm_bias_act_kernel(x_ref, w_ref, b_ref, o_ref, *, act):
    y = jnp.dot(x_ref[...], w_ref[...], preferred_element_type=_F32,
                precision=_PREC)
    y = y + b_ref[...]
    if act == "relu":
        y = jnp.maximum(y, 0.0)
    o_ref[...] = y


def _conv1(x, w, b):
    n, h, wd, c = x.shape
    o = w.shape[0]
    ho, wo = h // 2, wd // 2
    xp = jnp.pad(x, ((0, 0), (1, 1), (1, 1), (0, 0)))
    hp = (h + 2) // 2
    xr = xp.reshape(n, hp, 2, hp, 2, c)
    planes = [[xr[:, :, a, :, bb, :] for bb in (0, 1)] for a in (0, 1)]
    pieces = []
    for d in range(4):
        for e in range(4):
            p = planes[d % 2][e % 2]
            pieces.append(p[:, d // 2:d // 2 + ho, e // 2:e // 2 + wo, :])
    xcat = jnp.concatenate(pieces, axis=-1).reshape(n * ho * wo, 16 * c)
    wmat = jnp.transpose(w, (2, 3, 1, 0)).reshape(16 * c, o)
    m = n * ho * wo
    mb = m // 8
    out = pl.pallas_call(
        functools.partial(_mm_bias_act_kernel, act="relu"),
        grid=(8,),
        in_specs=[
            pl.BlockSpec((mb, 16 * c), lambda i: (i, 0)),
            pl.BlockSpec((16 * c, o), lambda i: (0, 0)),
            pl.BlockSpec((1, o), lambda i: (0, 0)),
        ],
        out_specs=pl.BlockSpec((mb, o), lambda i: (i, 0)),
        out_shape=jax.ShapeDtypeStruct((m, o), _F32),
    )(xcat, wmat, b.reshape(1, o))
    return out.reshape(n, ho, wo, o)


def _conv_stage_kernel(ee, eo, oe, oo, w_ref, b_ref, o_ref, *, ho, wo, cin, act):
    planes = ((ee, eo), (oe, oo))
    y = None
    for d in range(4):
        pieces = []
        for e in range(4):
            p = planes[d % 2][e % 2]
            sl = p[0, d // 2:d // 2 + ho, e // 2:e // 2 + wo, :]
            pieces.append(sl.reshape(ho * wo, cin))
        xcat = jnp.concatenate(pieces, axis=1)
        wg = w_ref[4 * d * cin:4 * (d + 1) * cin, :]
        part = jnp.dot(xcat, wg, preferred_element_type=_F32, precision=_PREC)
        y = part if y is None else y + part
    y = y + b_ref[...]
    if act == "relu":
        y = jnp.maximum(y, 0.0)
    o_ref[...] = y.reshape(1, ho, wo, y.shape[-1])


def _conv_stage(x, w, b, act):
    n, h, wd, c = x.shape
    o = w.shape[0]
    ho, wo = h // 2, wd // 2
    xp = jnp.pad(x, ((0, 0), (1, 1), (1, 1), (0, 0)))
    hp = (h + 2) // 2
    xr = xp.reshape(n, hp, 2, hp, 2, c)
    planes = [xr[:, :, a, :, bb, :] for a in (0, 1) for bb in (0, 1)]
    wmat = jnp.transpose(w, (2, 3, 1, 0)).reshape(16 * c, o)
    in_specs = [pl.BlockSpec((1, hp, hp, c), lambda i: (i, 0, 0, 0))
                for _ in range(4)]
    in_specs += [pl.BlockSpec((16 * c, o), lambda i: (0, 0)),
                 pl.BlockSpec((1, o), lambda i: (0, 0))]
    out = pl.pallas_call(
        functools.partial(_conv_stage_kernel, ho=ho, wo=wo, cin=c, act=act),
        grid=(n,),
        in_specs=in_specs,
        out_specs=pl.BlockSpec((1, ho, wo, o), lambda i: (i, 0, 0, 0)),
        out_shape=jax.ShapeDtypeStruct((n, ho, wo, o), _F32),
    )(*planes, wmat, b.reshape(1, o))
    return out


def _vq_kernel(z_ref, e_ref, q_ref, loss_ref, *, mb, k, ncodes):
    i = pl.program_id(0)
    zz = z_ref[...]
    ee = e_ref[...]
    scores = lax.dot_general(zz, ee, (((1,), (1,)), ((), ())),
                             preferred_element_type=_F32, precision=_PREC)
    ones = jnp.ones((1, k), dtype=_F32)
    esq = lax.dot_general(ones, ee * ee, (((1,), (1,)), ((), ())),
                          preferred_element_type=_F32, precision=_PREC)
    d = esq - 2.0 * scores
    dmin = jnp.min(d, axis=1, keepdims=True)
    iota = lax.broadcasted_iota(jnp.int32, (mb, ncodes), 1)
    idx = jnp.min(jnp.where(d == dmin, iota, ncodes), axis=1, keepdims=True)
    oh = (iota == idx).astype(_F32)
    q = jnp.dot(oh, ee, preferred_element_type=_F32, precision=_PREC)
    q_ref[...] = q
    diff = q - zz
    part = jnp.sum(diff * diff).reshape(1, 1)

    @pl.when(i == 0)
    def _():
        loss_ref[...] = jnp.zeros((1, 1), _F32)

    loss_ref[...] += part


def _vq(zflat, emb):
    m, k = zflat.shape
    ncodes = emb.shape[0]
    nblk = 4
    mb = m // nblk
    q, loss = pl.pallas_call(
        functools.partial(_vq_kernel, mb=mb, k=k, ncodes=ncodes),
        grid=(nblk,),
        in_specs=[pl.BlockSpec((mb, k), lambda i: (i, 0)),
                  pl.BlockSpec((ncodes, k), lambda i: (0, 0))],
        out_specs=[pl.BlockSpec((mb, k), lambda i: (i, 0)),
                   pl.BlockSpec((1, 1), lambda i: (0, 0))],
        out_shape=[jax.ShapeDtypeStruct((m, k), _F32),
                   jax.ShapeDtypeStruct((1, 1), _F32)],
    )(zflat, emb)
    vq_loss = (1.25 / (m * k)) * loss[0, 0]
    return q, vq_loss


def _deconv_stage_kernel(xp_ref, w00, w01, w10, w11, b_ref, o_ref,
                         *, rows, wo2, cin, cout, act):
    wrefs = ((w00, w01), (w10, w11))
    for py in range(2):
        for px in range(2):
            pieces = []
            for dy in range(2):
                for dx in range(2):
                    sl = xp_ref[0, py + dy:py + dy + rows,
                                px + dx:px + dx + wo2, :]
                    pieces.append(sl.reshape(rows * wo2, cin))
            xc = jnp.concatenate(pieces, axis=1)
            y = jnp.dot(xc, wrefs[py][px][...], preferred_element_type=_F32,
                        precision=_PREC)
            y = y + b_ref[...]
            if act == "relu":
                y = jnp.maximum(y, 0.0)
            elif act == "sigmoid":
                y = jax.nn.sigmoid(y)
            p = py * 2 + px
            o_ref[0, :, p * cout:(p + 1) * cout] = y


def _deconv_stage(x, w, b, act, row_tiles=1):
    n, hi, wi, c = x.shape
    o = w.shape[1]
    t = row_tiles
    r = hi // t
    xp = jnp.pad(x, ((0, 0), (1, 1), (1, 1), (0, 0)))
    if t > 1:
        xp = jnp.stack([xp[:, k * r:k * r + r + 2] for k in range(t)], axis=1)
        xp = xp.reshape(n * t, r + 2, wi + 2, c)
    wphase = []
    for py in range(2):
        for px in range(2):
            taps = []
            for dy in range(2):
                for dx in range(2):
                    taps.append(w[:, :, 3 - py - 2 * dy, 3 - px - 2 * dx])
            wphase.append(jnp.concatenate(taps, axis=0))
    in_specs = [pl.BlockSpec((1, r + 2, wi + 2, c), lambda i: (i, 0, 0, 0))]
    in_specs += [pl.BlockSpec((4 * c, o), lambda i: (0, 0)) for _ in range(4)]
    in_specs += [pl.BlockSpec((1, o), lambda i: (0, 0))]
    out = pl.pallas_call(
        functools.partial(_deconv_stage_kernel, rows=r, wo2=wi, cin=c,
                          cout=o, act=act),
        grid=(n * t,),
        in_specs=in_specs,
        out_specs=pl.BlockSpec((1, r * wi, 4 * o), lambda i: (i, 0, 0)),
        out_shape=jax.ShapeDtypeStruct((n * t, r * wi, 4 * o), _F32),
    )(xp, *wphase, b.reshape(1, o))
    rr = out.reshape(n, t, r, wi, 2, 2, o)
    rr = jnp.transpose(rr, (0, 1, 2, 4, 3, 5, 6)).reshape(n, 2 * hi, 2 * wi, o)
    return rr


def kernel(x, conv1_w, conv1_b, conv2_w, conv2_b, conv3_w, conv3_b,
           embeddings, deconv1_w, deconv1_b, deconv2_w, deconv2_b,
           deconv3_w, deconv3_b):
    n = x.shape[0]
    xh = jnp.transpose(x, (0, 2, 3, 1))
    z = _conv1(xh, conv1_w, conv1_b)
    z = _conv_stage(z, conv2_w, conv2_b, "relu")
    z = _conv_stage(z, conv3_w, conv3_b, "none")
    k = z.shape[-1]
    q, vq_loss = _vq(z.reshape(-1, k), embeddings)
    zq = q.reshape(n, 28, 28, k)
    h = _deconv_stage(zq, deconv1_w, deconv1_b, "relu")
    h = _deconv_stage(h, deconv2_w, deconv2_b, "relu")
    h = _deconv_stage(h, deconv3_w, deconv3_b, "sigmoid", row_tiles=4)
    xrec = jnp.transpose(h, (0, 3, 1, 2))
    return (xrec, vq_loss)


# speedup vs baseline: 1.3217x; 1.3217x over previous
import functools

import jax
import jax.numpy as jnp
from jax import lax
from jax.experimental import pallas as pl
from jax.experimental.pallas import tpu as pltpu

_PAR = pltpu.CompilerParams(dimension_semantics=("parallel",))

_PREC = lax.Precision.DEFAULT
_PREC_EXACT = lax.Precision.HIGHEST
_F32 = jnp.float32


def _mm_bias_act_kernel(x_ref, w_ref, b_ref, o_ref, *, act):
    y = jnp.dot(x_ref[...], w_ref[...], preferred_element_type=_F32,
                precision=_PREC)
    y = y + b_ref[...]
    if act == "relu":
        y = jnp.maximum(y, 0.0)
    o_ref[...] = y


def _conv1(x, w, b):
    n, h, wd, c = x.shape
    o = w.shape[0]
    ho, wo = h // 2, wd // 2
    xp = jnp.pad(x, ((0, 0), (1, 1), (1, 1), (0, 0)))
    hp = (h + 2) // 2
    xr = xp.reshape(n, hp, 2, hp, 2, c)
    planes = [[xr[:, :, a, :, bb, :] for bb in (0, 1)] for a in (0, 1)]
    pieces = []
    for d in range(4):
        for e in range(4):
            p = planes[d % 2][e % 2]
            pieces.append(p[:, d // 2:d // 2 + ho, e // 2:e // 2 + wo, :])
    xcat = jnp.concatenate(pieces, axis=-1).reshape(n * ho * wo, 16 * c)
    wmat = jnp.transpose(w, (2, 3, 1, 0)).reshape(16 * c, o)
    m = n * ho * wo
    mb = m // 8
    out = pl.pallas_call(
        functools.partial(_mm_bias_act_kernel, act="relu"),
        grid=(8,),
        in_specs=[
            pl.BlockSpec((mb, 16 * c), lambda i: (i, 0)),
            pl.BlockSpec((16 * c, o), lambda i: (0, 0)),
            pl.BlockSpec((1, o), lambda i: (0, 0)),
        ],
        out_specs=pl.BlockSpec((mb, o), lambda i: (i, 0)),
        out_shape=jax.ShapeDtypeStruct((m, o), _F32),
        compiler_params=_PAR,
    )(xcat, wmat, b.reshape(1, o))
    return out.reshape(n, ho, wo, o)


def _conv_stage_kernel(ee, eo, oe, oo, w_ref, b_ref, o_ref, *, ho, wo, cin, act):
    planes = ((ee, eo), (oe, oo))
    y = None
    for d in range(4):
        pieces = []
        for e in range(4):
            p = planes[d % 2][e % 2]
            sl = p[0, d // 2:d // 2 + ho, e // 2:e // 2 + wo, :]
            pieces.append(sl.reshape(ho * wo, cin))
        xcat = jnp.concatenate(pieces, axis=1)
        wg = w_ref[4 * d * cin:4 * (d + 1) * cin, :]
        part = jnp.dot(xcat, wg, preferred_element_type=_F32, precision=_PREC)
        y = part if y is None else y + part
    y = y + b_ref[...]
    if act == "relu":
        y = jnp.maximum(y, 0.0)
    o_ref[...] = y.reshape(1, ho, wo, y.shape[-1])


def _conv_stage(x, w, b, act):
    n, h, wd, c = x.shape
    o = w.shape[0]
    ho, wo = h // 2, wd // 2
    xp = jnp.pad(x, ((0, 0), (1, 1), (1, 1), (0, 0)))
    hp = (h + 2) // 2
    xr = xp.reshape(n, hp, 2, hp, 2, c)
    planes = [xr[:, :, a, :, bb, :] for a in (0, 1) for bb in (0, 1)]
    wmat = jnp.transpose(w, (2, 3, 1, 0)).reshape(16 * c, o)
    in_specs = [pl.BlockSpec((1, hp, hp, c), lambda i: (i, 0, 0, 0))
                for _ in range(4)]
    in_specs += [pl.BlockSpec((16 * c, o), lambda i: (0, 0)),
                 pl.BlockSpec((1, o), lambda i: (0, 0))]
    out = pl.pallas_call(
        functools.partial(_conv_stage_kernel, ho=ho, wo=wo, cin=c, act=act),
        grid=(n,),
        in_specs=in_specs,
        out_specs=pl.BlockSpec((1, ho, wo, o), lambda i: (i, 0, 0, 0)),
        out_shape=jax.ShapeDtypeStruct((n, ho, wo, o), _F32),
        compiler_params=_PAR,
    )(*planes, wmat, b.reshape(1, o))
    return out


def _vq_kernel(z_ref, e_ref, q_ref, loss_ref, *, mb, k, ncodes):
    zz = z_ref[...]
    ee = e_ref[...]
    scores = lax.dot_general(zz, ee, (((1,), (1,)), ((), ())),
                             preferred_element_type=_F32, precision=_PREC)
    ones = jnp.ones((1, k), dtype=_F32)
    esq = lax.dot_general(ones, ee * ee, (((1,), (1,)), ((), ())),
                          preferred_element_type=_F32, precision=_PREC)
    d = esq - 2.0 * scores
    dmin = jnp.min(d, axis=1, keepdims=True)
    iota = lax.broadcasted_iota(jnp.int32, (mb, ncodes), 1)
    idx = jnp.min(jnp.where(d == dmin, iota, ncodes), axis=1, keepdims=True)
    oh = (iota == idx).astype(_F32)
    q = jnp.dot(oh, ee, preferred_element_type=_F32, precision=_PREC_EXACT)
    q_ref[...] = q
    diff = q - zz
    loss_ref[...] = jnp.sum(diff * diff).reshape(1, 1, 1)


def _vq(zflat, emb):
    m, k = zflat.shape
    ncodes = emb.shape[0]
    nblk = 4
    mb = m // nblk
    q, loss = pl.pallas_call(
        functools.partial(_vq_kernel, mb=mb, k=k, ncodes=ncodes),
        grid=(nblk,),
        in_specs=[pl.BlockSpec((mb, k), lambda i: (i, 0)),
                  pl.BlockSpec((ncodes, k), lambda i: (0, 0))],
        out_specs=[pl.BlockSpec((mb, k), lambda i: (i, 0)),
                   pl.BlockSpec((1, 1, 1), lambda i: (i, 0, 0))],
        out_shape=[jax.ShapeDtypeStruct((m, k), _F32),
                   jax.ShapeDtypeStruct((nblk, 1, 1), _F32)],
        compiler_params=_PAR,
    )(zflat, emb)
    vq_loss = (1.25 / (m * k)) * jnp.sum(loss)
    return q, vq_loss


def _deconv_stage_kernel(xp_ref, w00, w01, w10, w11, b_ref, o_ref,
                         *, rows, wo2, cin, cout, act):
    wrefs = ((w00, w01), (w10, w11))
    for py in range(2):
        for px in range(2):
            pieces = []
            for dy in range(2):
                for dx in range(2):
                    sl = xp_ref[0, py + dy:py + dy + rows,
                                px + dx:px + dx + wo2, :]
                    pieces.append(sl.reshape(rows * wo2, cin))
            xc = jnp.concatenate(pieces, axis=1)
            y = jnp.dot(xc, wrefs[py][px][...], preferred_element_type=_F32,
                        precision=_PREC)
            y = y + b_ref[...]
            if act == "relu":
                y = jnp.maximum(y, 0.0)
            elif act == "sigmoid":
                y = jax.nn.sigmoid(y)
            p = py * 2 + px
            o_ref[0, :, p * cout:(p + 1) * cout] = y


def _deconv_stage(x, w, b, act, row_tiles=1):
    n, hi, wi, c = x.shape
    o = w.shape[1]
    t = row_tiles
    r = hi // t
    xp = jnp.pad(x, ((0, 0), (1, 1), (1, 1), (0, 0)))
    if t > 1:
        xp = jnp.stack([xp[:, k * r:k * r + r + 2] for k in range(t)], axis=1)
        xp = xp.reshape(n * t, r + 2, wi + 2, c)
    wphase = []
    for py in range(2):
        for px in range(2):
            taps = []
            for dy in range(2):
                for dx in range(2):
                    taps.append(w[:, :, 3 - py - 2 * dy, 3 - px - 2 * dx])
            wphase.append(jnp.concatenate(taps, axis=0))
    in_specs = [pl.BlockSpec((1, r + 2, wi + 2, c), lambda i: (i, 0, 0, 0))]
    in_specs += [pl.BlockSpec((4 * c, o), lambda i: (0, 0)) for _ in range(4)]
    in_specs += [pl.BlockSpec((1, o), lambda i: (0, 0))]
    out = pl.pallas_call(
        functools.partial(_deconv_stage_kernel, rows=r, wo2=wi, cin=c,
                          cout=o, act=act),
        grid=(n * t,),
        in_specs=in_specs,
        out_specs=pl.BlockSpec((1, r * wi, 4 * o), lambda i: (i, 0, 0)),
        out_shape=jax.ShapeDtypeStruct((n * t, r * wi, 4 * o), _F32),
        compiler_params=_PAR,
    )(xp, *wphase, b.reshape(1, o))
    rr = out.reshape(n, t, r, wi, 2, 2, o)
    rr = jnp.transpose(rr, (0, 1, 2, 4, 3, 5, 6)).reshape(n, 2 * hi, 2 * wi, o)
    return rr


def kernel(x, conv1_w, conv1_b, conv2_w, conv2_b, conv3_w, conv3_b,
           embeddings, deconv1_w, deconv1_b, deconv2_w, deconv2_b,
           deconv3_w, deconv3_b):
    n = x.shape[0]
    xh = jnp.transpose(x, (0, 2, 3, 1))
    z = _conv1(xh, conv1_w, conv1_b)
    z = _conv_stage(z, conv2_w, conv2_b, "relu")
    z = _conv_stage(z, conv3_w, conv3_b, "none")
    k = z.shape[-1]
    q, vq_loss = _vq(z.reshape(-1, k), embeddings)
    zq = q.reshape(n, 28, 28, k)
    h = _deconv_stage(zq, deconv1_w, deconv1_b, "relu")
    h = _deconv_stage(h, deconv2_w, deconv2_b, "relu")
    h = _deconv_stage(h, deconv3_w, deconv3_b, "sigmoid", row_tiles=4)
    xrec = jnp.transpose(h, (0, 3, 1, 2))
    return (xrec, vq_loss)


# speedup vs baseline: 1.5040x; 1.1380x over previous
import functools

import jax
import jax.numpy as jnp
from jax import lax
from jax.experimental import pallas as pl
from jax.experimental.pallas import tpu as pltpu
from jax.experimental.pallas import tpu_sc as plsc

_PAR = pltpu.CompilerParams(dimension_semantics=("parallel",))
_PREC = lax.Precision.DEFAULT
_F32 = jnp.float32


def _mm_bias_act_kernel(x_ref, w_ref, b_ref, o_ref, *, act):
    y = jnp.dot(x_ref[...], w_ref[...], preferred_element_type=_F32,
                precision=_PREC)
    y = y + b_ref[...]
    if act == "relu":
        y = jnp.maximum(y, 0.0)
    o_ref[...] = y


def _conv1(x, w, b):
    n, h, wd, c = x.shape
    o = w.shape[0]
    ho, wo = h // 2, wd // 2
    xp = jnp.pad(x, ((0, 0), (1, 1), (1, 1), (0, 0)))
    hp = (h + 2) // 2
    xr = xp.reshape(n, hp, 2, hp, 2, c)
    planes = [[xr[:, :, a, :, bb, :] for bb in (0, 1)] for a in (0, 1)]
    pieces = []
    for d in range(4):
        for e in range(4):
            p = planes[d % 2][e % 2]
            pieces.append(p[:, d // 2:d // 2 + ho, e // 2:e // 2 + wo, :])
    xcat = jnp.concatenate(pieces, axis=-1).reshape(n * ho * wo, 16 * c)
    wmat = jnp.transpose(w, (2, 3, 1, 0)).reshape(16 * c, o)
    m = n * ho * wo
    mb = m // 8
    out = pl.pallas_call(
        functools.partial(_mm_bias_act_kernel, act="relu"),
        grid=(8,),
        in_specs=[
            pl.BlockSpec((mb, 16 * c), lambda i: (i, 0)),
            pl.BlockSpec((16 * c, o), lambda i: (0, 0)),
            pl.BlockSpec((1, o), lambda i: (0, 0)),
        ],
        out_specs=pl.BlockSpec((mb, o), lambda i: (i, 0)),
        out_shape=jax.ShapeDtypeStruct((m, o), _F32),
        compiler_params=_PAR,
    )(xcat, wmat, b.reshape(1, o))
    return out.reshape(n, ho, wo, o)


def _k1_kernel(yq_ref, w2_ref, b2_ref, w3_ref, b3_ref, e_ref,
               idx_ref, loss_ref, q2_ref, *, g28, ncodes):
    q2_ref[0:1, :, :] = jnp.zeros((1, 30, 512), _F32)
    q2_ref[29:30, :, :] = jnp.zeros((1, 30, 512), _F32)
    q2_ref[:, 0:1, :] = jnp.zeros((30, 1, 512), _F32)
    q2_ref[:, 29:30, :] = jnp.zeros((30, 1, 512), _F32)
    m = g28 * g28
    for a in range(2):
        for bb in range(2):
            pieces = []
            for d in range(4):
                for e in range(4):
                    gy, gx = 2 * a + d - 1, 2 * bb + e - 1
                    by, sy = gy % 4, gy // 4
                    bx, sx = gx % 4, gx // 4
                    cb = (by * 4 + bx) * 64
                    sl = yq_ref[0, sy + 1:sy + 1 + g28,
                                sx + 1:sx + 1 + g28, cb:cb + 64]
                    pieces.append(sl.reshape(m, 64))
            xc = jnp.concatenate(pieces, axis=1)
            y = jnp.dot(xc, w2_ref[...], preferred_element_type=_F32,
                        precision=_PREC)
            y = jnp.maximum(y + b2_ref[...], 0.0)
            cb = (a * 2 + bb) * 128
            q2_ref[1:29, 1:29, cb:cb + 128] = y.reshape(g28, g28, 128)
    pieces = []
    for d in range(4):
        for e in range(4):
            py, sy = (d - 1) % 2, (d - 1) // 2
            px, sx = (e - 1) % 2, (e - 1) // 2
            cb = (py * 2 + px) * 128
            sl = q2_ref[sy + 1:sy + 1 + g28, sx + 1:sx + 1 + g28,
                        cb:cb + 128]
            pieces.append(sl.reshape(m, 128))
    xc = jnp.concatenate(pieces, axis=1)
    z = jnp.dot(xc, w3_ref[...], preferred_element_type=_F32,
                precision=_PREC) + b3_ref[...]
    ee = e_ref[...]
    scores = lax.dot_general(z, ee, (((1,), (1,)), ((), ())),
                             preferred_element_type=_F32, precision=_PREC)
    ones = jnp.ones((1, z.shape[1]), dtype=_F32)
    esq = lax.dot_general(ones, ee * ee, (((1,), (1,)), ((), ())),
                          preferred_element_type=_F32, precision=_PREC)
    dmat = esq - 2.0 * scores
    dmin = jnp.min(dmat, axis=1, keepdims=True)
    iota = lax.broadcasted_iota(jnp.int32, (m, ncodes), 1)
    idx = jnp.min(jnp.where(dmat == dmin, iota, ncodes), axis=1,
                  keepdims=True)
    idx_ref[...] = idx
    zsq = jnp.sum(z * z, axis=1, keepdims=True)
    loss_ref[...] = jnp.sum(dmin + zsq).reshape(1, 1, 1)


def _encode_vq(y1, conv2_w, conv2_b, conv3_w, conv3_b, emb):
    n = y1.shape[0]
    y1p = jnp.pad(y1, ((0, 0), (4, 4), (4, 4), (0, 0)))
    qy1 = y1p.reshape(n, 30, 4, 30, 4, 64)
    qy1 = jnp.transpose(qy1, (0, 1, 3, 2, 4, 5)).reshape(n, 30, 30, 1024)
    w2 = jnp.transpose(conv2_w, (2, 3, 1, 0)).reshape(1024, 128)
    w3 = jnp.transpose(conv3_w, (2, 3, 1, 0)).reshape(2048, 256)
    ncodes = emb.shape[0]
    m = 784
    idx, loss = pl.pallas_call(
        functools.partial(_k1_kernel, g28=28, ncodes=ncodes),
        grid=(n,),
        in_specs=[
            pl.BlockSpec((1, 30, 30, 1024), lambda i: (i, 0, 0, 0)),
            pl.BlockSpec((1024, 128), lambda i: (0, 0)),
            pl.BlockSpec((1, 128), lambda i: (0, 0)),
            pl.BlockSpec((2048, 256), lambda i: (0, 0)),
            pl.BlockSpec((1, 256), lambda i: (0, 0)),
            pl.BlockSpec((ncodes, 256), lambda i: (0, 0)),
        ],
        out_specs=[
            pl.BlockSpec((m, 1), lambda i: (i, 0)),
            pl.BlockSpec((1, 1, 1), lambda i: (i, 0, 0)),
        ],
        out_shape=[
            jax.ShapeDtypeStruct((n * m, 1), jnp.int32),
            jax.ShapeDtypeStruct((n, 1, 1), _F32),
        ],
        scratch_shapes=[pltpu.VMEM((30, 30, 512), _F32)],
        compiler_params=_PAR,
    )(qy1, w2, conv2_b.reshape(1, 128), w3, conv3_b.reshape(1, 256), emb)
    vq_loss = (1.25 / (n * m * 256)) * jnp.sum(loss)
    return idx, vq_loss


def _sc_gather(emb, idx_flat):
    bsz = idx_flat.shape[0]
    dd = emb.shape[1]
    nw = 32
    bw = bsz // nw
    mesh = plsc.VectorSubcoreMesh(core_axis_name="c", subcore_axis_name="s")

    @functools.partial(
        pl.kernel, mesh=mesh,
        out_type=jax.ShapeDtypeStruct((bsz, dd), _F32),
        scratch_types=[
            pltpu.VMEM((bw,), jnp.int32),
            pltpu.VMEM((bw, dd), _F32),
            pltpu.SemaphoreType.DMA,
        ],
    )
    def kern(table_hbm, idx_hbm, out_hbm, idx_v, rows_v, sem):
        wid = lax.axis_index("s") * 2 + lax.axis_index("c")
        base = wid * bw
        pltpu.sync_copy(idx_hbm.at[pl.ds(base, bw)], idx_v)
        pltpu.async_copy(table_hbm.at[idx_v], rows_v, sem).wait()
        pltpu.sync_copy(rows_v, out_hbm.at[pl.ds(base, bw)])

    return kern(emb, idx_flat)


def _dctaps(g, p):
    taps = []
    for dyp in range(2):
        d = (g + 1) % 2 + 2 * dyp
        h = (g - d + 1) // 2
        taps.append((h % p, h // p, d))
    return taps


def _k2_kernel(q_ref, w1_ref, b1_ref, w2_ref, b2_ref, w3_ref, b3_ref,
               o_ref, qs_ref, d1_ref, d2_ref, *, g28):
    m = g28 * g28
    qs_ref[0:1, :, :] = jnp.zeros((1, 30, 256), _F32)
    qs_ref[29:30, :, :] = jnp.zeros((1, 30, 256), _F32)
    qs_ref[:, 0:1, :] = jnp.zeros((30, 1, 256), _F32)
    qs_ref[:, 29:30, :] = jnp.zeros((30, 1, 256), _F32)
    qs_ref[1:29, 1:29, :] = q_ref[0]
    d1_ref[0:1, :, :] = jnp.zeros((1, 30, 512), _F32)
    d1_ref[29:30, :, :] = jnp.zeros((1, 30, 512), _F32)
    d1_ref[:, 0:1, :] = jnp.zeros((30, 1, 512), _F32)
    d1_ref[:, 29:30, :] = jnp.zeros((30, 1, 512), _F32)
    for py in range(2):
        for px in range(2):
            pieces = []
            for ty in _dctaps(py, 1):
                for tx in _dctaps(px, 1):
                    sl = qs_ref[ty[1] + 1:ty[1] + 1 + g28,
                                tx[1] + 1:tx[1] + 1 + g28, :]
                    pieces.append(sl.reshape(m, 256))
            xc = jnp.concatenate(pieces, axis=1)
            cb = (py * 2 + px) * 128
            y = jnp.dot(xc, w1_ref[:, cb:cb + 128],
                        preferred_element_type=_F32, precision=_PREC)
            y = jnp.maximum(y + b1_ref[...], 0.0)
            d1_ref[1:29, 1:29, cb:cb + 128] = y.reshape(g28, g28, 128)
    d2_ref[0:1, :, :] = jnp.zeros((1, 30, 1024), _F32)
    d2_ref[29:30, :, :] = jnp.zeros((1, 30, 1024), _F32)
    d2_ref[:, 0:1, :] = jnp.zeros((30, 1, 1024), _F32)
    d2_ref[:, 29:30, :] = jnp.zeros((30, 1, 1024), _F32)
    for gy in range(4):
        for gx in range(4):
            pieces = []
            for (pin_y, sy, _dy) in _dctaps(gy, 2):
                for (pin_x, sx, _dx) in _dctaps(gx, 2):
                    cb = (pin_y * 2 + pin_x) * 128
                    sl = d1_ref[sy + 1:sy + 1 + g28,
                                sx + 1:sx + 1 + g28, cb:cb + 128]
                    pieces.append(sl.reshape(m, 128))
            xc = jnp.concatenate(pieces, axis=1)
            cb = (gy * 4 + gx) * 64
            y = jnp.dot(xc, w2_ref[:, cb:cb + 64],
                        preferred_element_type=_F32, precision=_PREC)
            y = jnp.maximum(y + b2_ref[...], 0.0)
            d2_ref[1:29, 1:29, cb:cb + 64] = y.reshape(g28, g28, 64)
    for gy in range(8):
        for gx in range(8):
            pieces = []
            for (pin_y, sy, _dy) in _dctaps(gy, 4):
                for (pin_x, sx, _dx) in _dctaps(gx, 4):
                    cb = (pin_y * 4 + pin_x) * 64
                    sl = d2_ref[sy + 1:sy + 1 + g28,
                                sx + 1:sx + 1 + g28, cb:cb + 64]
                    pieces.append(sl.reshape(m, 64))
            xc = jnp.concatenate(pieces, axis=1)
            cb = (gy * 8 + gx) * 3
            y = jnp.dot(xc, w3_ref[:, cb:cb + 3],
                        preferred_element_type=_F32, precision=_PREC)
            y = jax.nn.sigmoid(y + b3_ref[...])
            o_ref[0, :, cb:cb + 3] = y


def _decode(q, deconv1_w, deconv1_b, deconv2_w, deconv2_b,
            deconv3_w, deconv3_b):
    n = q.shape[0]

    def pack_w(w, pcount, cin_blk, tap_cin):
        cols = []
        for gy in range(pcount):
            for gx in range(pcount):
                taps = []
                for (_, _, dy) in _dctaps(gy, max(pcount // 2, 1)):
                    for (_, _, dx) in _dctaps(gx, max(pcount // 2, 1)):
                        taps.append(w[:, :, dy, dx])
                cols.append(jnp.concatenate(taps, axis=0))
        return jnp.concatenate(cols, axis=1)

    w1 = pack_w(deconv1_w, 2, 256, 4)
    w2 = pack_w(deconv2_w, 4, 128, 4)
    w3 = pack_w(deconv3_w, 8, 64, 4)
    m = 784
    out = pl.pallas_call(
        functools.partial(_k2_kernel, g28=28),
        grid=(n,),
        in_specs=[
            pl.BlockSpec((1, 28, 28, 256), lambda i: (i, 0, 0, 0)),
            pl.BlockSpec((1024, 512), lambda i: (0, 0)),
            pl.BlockSpec((1, 128), lambda i: (0, 0)),
            pl.BlockSpec((512, 1024), lambda i: (0, 0)),
            pl.BlockSpec((1, 64), lambda i: (0, 0)),
            pl.BlockSpec((256, 192), lambda i: (0, 0)),
            pl.BlockSpec((1, 3), lambda i: (0, 0)),
        ],
        out_specs=pl.BlockSpec((1, m, 192), lambda i: (i, 0, 0)),
        out_shape=jax.ShapeDtypeStruct((n, m, 192), _F32),
        scratch_shapes=[pltpu.VMEM((30, 30, 256), _F32),
                        pltpu.VMEM((30, 30, 512), _F32),
                        pltpu.VMEM((30, 30, 1024), _F32)],
        compiler_params=_PAR,
    )(q, w1, deconv1_b.reshape(1, 128), w2, deconv2_b.reshape(1, 64),
      w3, deconv3_b.reshape(1, 3))
    r = out.reshape(n, 28, 28, 8, 8, 3)
    r = jnp.transpose(r, (0, 5, 1, 3, 2, 4)).reshape(n, 3, 224, 224)
    return r


def kernel(x, conv1_w, conv1_b, conv2_w, conv2_b, conv3_w, conv3_b,
           embeddings, deconv1_w, deconv1_b, deconv2_w, deconv2_b,
           deconv3_w, deconv3_b):
    n = x.shape[0]
    xh = jnp.transpose(x, (0, 2, 3, 1))
    y1 = _conv1(xh, conv1_w, conv1_b)
    idx, vq_loss = _encode_vq(y1, conv2_w, conv2_b, conv3_w, conv3_b,
                              embeddings)
    m = n * 784
    mpad = ((m + 255) // 256) * 256
    idx_flat = jnp.pad(idx.reshape(m), (0, mpad - m))
    q = _sc_gather(embeddings, idx_flat)[:m]
    q = q.reshape(n, 28, 28, embeddings.shape[1])
    xrec = _decode(q, deconv1_w, deconv1_b, deconv2_w, deconv2_b,
                   deconv3_w, deconv3_b)
    return (xrec, vq_loss)
